# Initial kernel scaffold; baseline (speedup 1.0000x reference)
import jax
import jax.numpy as jnp
from jax.experimental import pallas as pl


def kernel(input, weights):
    raise NotImplementedError("write your pallas kernel here")


# speedup vs baseline: 3.1113x; 3.1113x over previous
import functools

import jax
import jax.numpy as jnp
from jax import lax
from jax.experimental import pallas as pl
from jax.experimental.pallas import tpu as pltpu
from jax.experimental.pallas import tpu_sc as plsc

B = 16384
S = 50
D = 64
NC = 2
NS = 16
NW = NC * NS
RPW = B // NW
PPW = RPW * S
CH = 128
NJ = PPW // CH
GROUPS = RPW // 16


def _build():
    mesh = plsc.VectorSubcoreMesh(core_axis_name="c", subcore_axis_name="s")

    @functools.partial(
        pl.kernel,
        mesh=mesh,
        compiler_params=pltpu.CompilerParams(
            needs_layout_passes=False, use_tc_tiling_on_sc=False),
        out_type=jax.ShapeDtypeStruct((B * S, D), jnp.float32),
        scratch_types=[
            pltpu.VMEM((PPW,), jnp.int32),
            pltpu.VMEM((NJ, CH), jnp.int32),
            pltpu.VMEM((2, CH, D), jnp.float32),
            pltpu.SemaphoreType.DMA,
            pltpu.SemaphoreType.DMA,
        ],
    )
    def emb_kernel(in_hbm, w_hbm, out_hbm, tok_ref, idx_ref, rows_ref,
                   sem0, sem1):
        wid = lax.axis_index("s") * NC + lax.axis_index("c")
        row0 = wid * RPW
        flat0 = row0 * S

        pltpu.sync_copy(in_hbm.at[pl.ds(flat0, PPW)], tok_ref)

        iota = lax.iota(jnp.int32, 16)

        def pos_body(g, carry):
            rows16 = g * 16 + iota
            pbase = rows16 * S
            running = jnp.zeros((16,), jnp.int32)
            for s in range(S):
                tok = plsc.load_gather(tok_ref, [pbase + s])
                m = tok != 0
                running = running + m.astype(jnp.int32)
                posv = jnp.where(m, running, 0)
                p = pbase + s
                plsc.store_scatter(
                    idx_ref,
                    [lax.shift_right_logical(p, 7),
                     lax.bitwise_and(p, CH - 1)],
                    posv)
            return carry

        lax.fori_loop(0, GROUPS, pos_body, 0)

        out0 = row0 * S
        sems = (sem0, sem1)

        pltpu.async_copy(w_hbm.at[idx_ref.at[0]], rows_ref.at[0], sem0)

        def stream_body(i, carry):
            for b in range(2):
                j = 2 * i + b
                nj = j + 1
                ob = 1 - b
                if b == 0:
                    pltpu.async_copy(
                        w_hbm.at[idx_ref.at[nj]], rows_ref.at[ob], sems[ob])
                else:
                    @pl.when(nj < NJ)
                    def _():
                        pltpu.async_copy(
                            w_hbm.at[idx_ref.at[nj]], rows_ref.at[ob],
                            sems[ob])
                pltpu.make_async_copy(
                    w_hbm.at[idx_ref.at[j]], rows_ref.at[b], sems[b]).wait()
                pltpu.sync_copy(
                    rows_ref.at[b], out_hbm.at[pl.ds(out0 + j * CH, CH)])
            return carry

        lax.fori_loop(0, NJ // 2, stream_body, 0)

    return emb_kernel


_EMB = _build()


@jax.jit
def kernel(input, weights):
    out = _EMB(input.reshape(B * S), weights)
    return out.reshape(B, S, D)
